# baseline (speedup 1.0000x reference)
import jax
import jax.numpy as jnp
from jax.experimental import pallas as pl
from jax.experimental.pallas import tpu as pltpu

_CHUNKS = 4


def _k3d(z_ref, w_ref, b_ref, o_ref):
    t = z_ref.shape[0]
    zb = z_ref[...].reshape(t * 8, z_ref.shape[2])
    acc = jnp.dot(zb, w_ref[...], preferred_element_type=jnp.float32)
    acc = acc + b_ref[...]
    o_ref[...] = acc.astype(o_ref.dtype).reshape(t, 8, o_ref.shape[2])


def _run_chunk(zc, w, b, tile):
    rows = zc.shape[0]
    in_dim = zc.shape[2]
    out_dim = w.shape[1]
    steps = pl.cdiv(rows, tile)
    return pl.pallas_call(
        _k3d,
        out_shape=jax.ShapeDtypeStruct((rows, 8, out_dim), zc.dtype),
        grid=(steps,),
        in_specs=[
            pl.BlockSpec((tile, 8, in_dim), lambda i: (i, 0, 0)),
            pl.BlockSpec((in_dim, out_dim), lambda i: (0, 0)),
            pl.BlockSpec((1, out_dim), lambda i: (0, 0)),
        ],
        out_specs=pl.BlockSpec((tile, 8, out_dim), lambda i: (i, 0, 0)),
        compiler_params=pltpu.CompilerParams(
            dimension_semantics=("parallel",),
            vmem_limit_bytes=60 * 1024 * 1024,
        ),
    )(zc, w, b)


def kernel(z, w12, b_eff, w_bd, b_bd):
    B, in_dim = z.shape
    out_dim = w12.shape[1]
    b = b_eff.reshape(1, out_dim)

    rows = B // 8
    chunks = _CHUNKS if rows % (_CHUNKS * 8) == 0 else 1
    crows = rows // chunks
    tile = min(2048, crows)

    outs = []
    for c in range(chunks):
        zc = jax.lax.slice_in_dim(z, c * crows * 8, (c + 1) * crows * 8, axis=0)
        outs.append(_run_chunk(zc.reshape(crows, 8, in_dim), w12, b, tile))
    out = outs[0] if chunks == 1 else jnp.concatenate(outs, axis=0)
    return out.reshape(B, out_dim)


# speedup vs baseline: 2.0670x; 2.0670x over previous
import jax
import jax.numpy as jnp
from jax.experimental import pallas as pl
from jax.experimental.pallas import tpu as pltpu


def _k3d(z_ref, w_ref, b_ref, o_ref):
    t = z_ref.shape[0]
    zb = z_ref[...].reshape(t * 8, z_ref.shape[2])
    acc = jnp.dot(zb, w_ref[...], preferred_element_type=jnp.float32)
    acc = acc + b_ref[...]
    o_ref[...] = acc.astype(o_ref.dtype).reshape(t, 8, o_ref.shape[2])


def kernel(z, w12, b_eff, w_bd, b_bd):
    B, in_dim = z.shape
    out_dim = w12.shape[1]
    b = b_eff.reshape(1, out_dim)

    zv = z.reshape(B // 8, 8, in_dim)
    rows = B // 8
    tile = 1024
    steps = pl.cdiv(rows, tile)
    out = pl.pallas_call(
        _k3d,
        out_shape=jax.ShapeDtypeStruct((rows, 8, out_dim), z.dtype),
        grid=(steps,),
        in_specs=[
            pl.BlockSpec((tile, 8, in_dim), lambda i: (i, 0, 0)),
            pl.BlockSpec((in_dim, out_dim), lambda i: (0, 0)),
            pl.BlockSpec((1, out_dim), lambda i: (0, 0)),
        ],
        out_specs=pl.BlockSpec((tile, 8, out_dim), lambda i: (i, 0, 0)),
        compiler_params=pltpu.CompilerParams(
            dimension_semantics=("parallel",),
            vmem_limit_bytes=60 * 1024 * 1024,
        ),
    )(zv, w12, b)

    return out.reshape(B, out_dim)


# speedup vs baseline: 2.0865x; 1.0094x over previous
import jax
import jax.numpy as jnp
from jax.experimental import pallas as pl
from jax.experimental.pallas import tpu as pltpu

_TILE = 2048


def _k_manual(z_hbm, w_ref, b_ref, o_hbm, zbuf, obuf, insem, outsem):
    i = pl.program_id(0)
    steps = pl.num_programs(0)
    slot = jax.lax.rem(i, 2)
    nslot = jax.lax.rem(i + 1, 2)

    @pl.when(i == 0)
    def _():
        pltpu.make_async_copy(
            z_hbm.at[pl.ds(0, _TILE)], zbuf.at[0], insem.at[0]
        ).start()

    @pl.when(i + 1 < steps)
    def _():
        pltpu.make_async_copy(
            z_hbm.at[pl.ds((i + 1) * _TILE, _TILE)], zbuf.at[nslot], insem.at[nslot]
        ).start()

    pltpu.make_async_copy(
        z_hbm.at[pl.ds(i * _TILE, _TILE)], zbuf.at[slot], insem.at[slot]
    ).wait()

    @pl.when(i >= 2)
    def _():
        pltpu.make_async_copy(obuf.at[slot], obuf.at[slot], outsem.at[slot]).wait()

    zb = zbuf[slot].reshape(_TILE * 8, zbuf.shape[3])
    acc = jnp.dot(zb, w_ref[...], preferred_element_type=jnp.float32)
    acc = acc + b_ref[...]
    obuf[slot] = acc.astype(obuf.dtype).reshape(_TILE, 8, obuf.shape[3])

    pltpu.make_async_copy(
        obuf.at[slot], o_hbm.at[pl.ds(i * _TILE, _TILE)], outsem.at[slot]
    ).start()

    @pl.when(i == steps - 1)
    def _():
        pltpu.make_async_copy(obuf.at[slot], obuf.at[slot], outsem.at[slot]).wait()

    @pl.when(i == steps - 1)
    def _():
        pltpu.make_async_copy(obuf.at[nslot], obuf.at[nslot], outsem.at[nslot]).wait()


def kernel(z, w12, b_eff, w_bd, b_bd):
    B, in_dim = z.shape
    out_dim = w12.shape[1]
    b = b_eff.reshape(1, out_dim)

    rows = B // 8
    zv = z.reshape(rows, 8, in_dim)
    steps = rows // _TILE
    out = pl.pallas_call(
        _k_manual,
        out_shape=jax.ShapeDtypeStruct((rows, 8, out_dim), z.dtype),
        grid=(steps,),
        in_specs=[
            pl.BlockSpec(memory_space=pl.ANY),
            pl.BlockSpec((in_dim, out_dim), lambda i: (0, 0)),
            pl.BlockSpec((1, out_dim), lambda i: (0, 0)),
        ],
        out_specs=pl.BlockSpec(memory_space=pl.ANY),
        scratch_shapes=[
            pltpu.VMEM((2, _TILE, 8, in_dim), jnp.float32),
            pltpu.VMEM((2, _TILE, 8, out_dim), jnp.float32),
            pltpu.SemaphoreType.DMA((2,)),
            pltpu.SemaphoreType.DMA((2,)),
        ],
        compiler_params=pltpu.CompilerParams(
            dimension_semantics=("arbitrary",),
            vmem_limit_bytes=60 * 1024 * 1024,
        ),
    )(zv, w12, b)

    return out.reshape(B, out_dim)
